# baseline (speedup 1.0000x reference)
import jax
import jax.numpy as jnp
from jax.experimental import pallas as pl

N = 10000
E = 320000
D = 128
L = 3
HEADS = 4
HID = 64
NC = 10
G = 64
CAT = L * D
M_ATT = 100000
EPS = 1e-4


def _gmp(v, seg, num):
    s = jax.ops.segment_sum(v, seg, num_segments=num)
    cnt = jnp.maximum(jax.ops.segment_sum(jnp.ones(seg.shape, jnp.float32), seg, num_segments=num), 1.0)
    if s.ndim > 1:
        cnt = cnt[:, None]
    return s / cnt


def _gcn(h, src, dst, W, b):
    n = h.shape[0]
    loop = jnp.arange(n, dtype=src.dtype)
    s = jnp.concatenate([src, loop])
    d = jnp.concatenate([dst, loop])
    deg = jax.ops.segment_sum(jnp.ones(d.shape, jnp.float32), d, num_segments=n)
    dis = 1.0 / jnp.sqrt(jnp.maximum(deg, 1.0))
    agg = jax.ops.segment_sum(h[s] * (dis[s] * dis[d])[:, None], d, num_segments=n)
    return agg @ W + b


def _bn(h, g, b):
    m = h.mean(axis=0)
    v = h.var(axis=0)
    return (h - m) / jnp.sqrt(v + 1e-5) * g + b


def kernel(x, edge_index, neg_edge_index, batch, mask, W_fc, b_fc, W_conv0, b_conv0, gamma0, beta0, W_conv1, b_conv1, gamma1, beta1, W_conv2, b_conv2, gamma2, beta2, W_feat, W_fc1, b_fc1, W_fc2, b_fc2, phi):
    mi = jnp.nonzero(mask, size=M_ATT, fill_value=0)
    midx0, midx1 = mi[0], mi[1]
    mvalid = jnp.arange(M_ATT) < jnp.sum(mask)
    ng = G
    convs = [(W_conv0, b_conv0), (W_conv1, b_conv1), (W_conv2, b_conv2)]
    bns = [(gamma0, beta0), (gamma1, beta1), (gamma2, beta2)]

    h = x @ W_fc + b_fc
    hlist = []
    for (Wc, bc), (g, bt) in zip(convs, bns):
        h = _bn(h, g, bt)
        h = jax.nn.relu(_gcn(h, edge_index[0], edge_index[1], Wc, bc))
        hlist.append(h)
    hx = jnp.concatenate(hlist, axis=-1)

    def dec(ei):
        return jax.nn.sigmoid(jnp.sum(hx[ei[0]] * hx[ei[1]], axis=1))

    pos = jnp.sum(_gmp(-jnp.log(EPS + dec(edge_index)), batch[edge_index[0]], ng))
    neg = jnp.sum(_gmp(-jnp.log(EPS + 1.0 - dec(neg_edge_index)), batch[neg_edge_index[0]], ng))
    Lrc = (pos + neg) / ng

    xh = (hx @ W_feat).reshape(-1, HEADS, HID)
    cx = jnp.concatenate([xh[midx0], xh[midx1]], axis=-1).transpose(1, 0, 2)
    wp = jax.nn.leaky_relu(jnp.matmul(cx, phi)).transpose(1, 0, 2).reshape(-1, HEADS)
    w = jax.nn.sigmoid(wp)
    xm = (xh[midx1] * w[:, :, None]).reshape(-1, HEADS * HID) * mvalid[:, None].astype(jnp.float32)
    xs = jax.ops.segment_sum(xm, midx0, num_segments=x.shape[0])
    hc = jax.nn.leaky_relu(xs @ W_fc1 + b_fc1)
    hc = hc @ W_fc2 + b_fc2
    tmp = jnp.exp(hc - jnp.max(hc, axis=-1, keepdims=True)) + EPS
    preds = tmp / jnp.sum(tmp, axis=-1, keepdims=True)
    yp = _gmp(preds, batch, ng)
    return (jnp.log(yp), Lrc, preds)


# speedup vs baseline: 1.5701x; 1.5701x over previous
import functools

import jax
import jax.numpy as jnp
from jax import lax
from jax.experimental import pallas as pl
from jax.experimental.pallas import tpu as pltpu
from jax.experimental.pallas import tpu_sc as plsc

N = 10000
E = 320000
D = 128
HEADS = 4
HID = 64
NCLS = 10
G = 64
CAT = 3 * D
M_ATT = 100000
EPS = 1e-4

BR = 1000
GRID_N = N // BR
EB = 2500
EBR = 2500
GRID_E = EB // EBR


def _fc_body(x_ref, w_ref, b_ref, h_ref, st_ref):
    i = pl.program_id(0)
    h = jnp.dot(x_ref[...], w_ref[...], preferred_element_type=jnp.float32)
    h = h + b_ref[...]
    h_ref[...] = h

    @pl.when(i == 0)
    def _():
        st_ref[...] = jnp.zeros_like(st_ref)

    st_ref[...] += jnp.stack([jnp.sum(h, axis=0), jnp.sum(h * h, axis=0)])


def _tc_fc(x, W, b):
    return pl.pallas_call(
        _fc_body,
        grid=(GRID_N,),
        in_specs=[
            pl.BlockSpec((BR, D), lambda i: (i, 0)),
            pl.BlockSpec((D, D), lambda i: (0, 0)),
            pl.BlockSpec((1, D), lambda i: (0, 0)),
        ],
        out_specs=[
            pl.BlockSpec((BR, D), lambda i: (i, 0)),
            pl.BlockSpec((2, D), lambda i: (0, 0)),
        ],
        out_shape=[
            jax.ShapeDtypeStruct((N, D), jnp.float32),
            jax.ShapeDtypeStruct((2, D), jnp.float32),
        ],
    )(x, W, b.reshape(1, D))


def _deg_body(p_ref, dis_ref):
    deg = 1.0 + jnp.sum(p_ref[...], axis=0, keepdims=True)
    dis_ref[...] = lax.rsqrt(jnp.maximum(deg, 1.0))


def _tc_deg(partials):
    return pl.pallas_call(
        _deg_body,
        grid=(1,),
        in_specs=[pl.BlockSpec((1, N), lambda i: (0, 0))],
        out_specs=pl.BlockSpec((1, N), lambda i: (0, 0)),
        out_shape=jax.ShapeDtypeStruct((1, N), jnp.float32),
    )(partials)


def _bnu_body(h_ref, st_ref, g_ref, bt_ref, dis_ref, u_ref):
    mean = st_ref[0:1, :] / N
    var = st_ref[1:2, :] / N - mean * mean
    rstd = lax.rsqrt(var + 1e-5)
    hb = (h_ref[...] - mean) * (rstd * g_ref[...]) + bt_ref[...]
    u_ref[...] = hb * dis_ref[...]


def _tc_bnu(h, stats, gamma, beta, dis):
    return pl.pallas_call(
        _bnu_body,
        grid=(GRID_N,),
        in_specs=[
            pl.BlockSpec((BR, D), lambda i: (i, 0)),
            pl.BlockSpec((2, D), lambda i: (0, 0)),
            pl.BlockSpec((1, D), lambda i: (0, 0)),
            pl.BlockSpec((1, D), lambda i: (0, 0)),
            pl.BlockSpec((BR, 1), lambda i: (i, 0)),
        ],
        out_specs=pl.BlockSpec((BR, D), lambda i: (i, 0)),
        out_shape=jax.ShapeDtypeStruct((N, D), jnp.float32),
    )(h, stats, gamma.reshape(1, D), beta.reshape(1, D), dis)


def _mm_body(p_ref, u_ref, dis_ref, w_ref, b_ref, h_ref, st_ref):
    i = pl.program_id(0)
    agg = (p_ref[...] + u_ref[...]) * dis_ref[...]
    h = jnp.dot(agg, w_ref[...], preferred_element_type=jnp.float32)
    h = jnp.maximum(h + b_ref[...], 0.0)
    h_ref[...] = h

    @pl.when(i == 0)
    def _():
        st_ref[...] = jnp.zeros_like(st_ref)

    st_ref[...] += jnp.stack([jnp.sum(h, axis=0), jnp.sum(h * h, axis=0)])


def _tc_mm(p, u, dis, W, b):
    return pl.pallas_call(
        _mm_body,
        grid=(GRID_N,),
        in_specs=[
            pl.BlockSpec((BR, D), lambda i: (i, 0)),
            pl.BlockSpec((BR, D), lambda i: (i, 0)),
            pl.BlockSpec((BR, 1), lambda i: (i, 0)),
            pl.BlockSpec((D, D), lambda i: (0, 0)),
            pl.BlockSpec((1, D), lambda i: (0, 0)),
        ],
        out_specs=[
            pl.BlockSpec((BR, D), lambda i: (i, 0)),
            pl.BlockSpec((2, D), lambda i: (0, 0)),
        ],
        out_shape=[
            jax.ShapeDtypeStruct((N, D), jnp.float32),
            jax.ShapeDtypeStruct((2, D), jnp.float32),
        ],
    )(p, u, dis, W, b.reshape(1, D))


def _feat_body(hx_ref, wf_ref, pa_ref, pb_ref, xh_ref, a_ref, b_ref):
    xh = jnp.dot(hx_ref[...], wf_ref[...], preferred_element_type=jnp.float32)
    xh_ref[...] = xh
    a_ref[...] = jnp.dot(xh, pa_ref[...], preferred_element_type=jnp.float32)
    b_ref[...] = jnp.dot(xh, pb_ref[...], preferred_element_type=jnp.float32)


def _tc_feat(hx, W_feat, PhiA, PhiB):
    HH = HEADS * HID
    return pl.pallas_call(
        _feat_body,
        grid=(GRID_N,),
        in_specs=[
            pl.BlockSpec((BR, CAT), lambda i: (i, 0)),
            pl.BlockSpec((CAT, HH), lambda i: (0, 0)),
            pl.BlockSpec((HH, HEADS), lambda i: (0, 0)),
            pl.BlockSpec((HH, HEADS), lambda i: (0, 0)),
        ],
        out_specs=[
            pl.BlockSpec((BR, HH), lambda i: (i, 0)),
            pl.BlockSpec((BR, HEADS), lambda i: (i, 0)),
            pl.BlockSpec((BR, HEADS), lambda i: (i, 0)),
        ],
        out_shape=[
            jax.ShapeDtypeStruct((N, HH), jnp.float32),
            jax.ShapeDtypeStruct((N, HEADS), jnp.float32),
            jax.ShapeDtypeStruct((N, HEADS), jnp.float32),
        ],
    )(hx, W_feat, PhiA, PhiB)


def _loss_body(tp_ref, tn_ref, sp_ref, sn_ref, sums_ref, cnts_ref, lrc_ref):
    i = pl.program_id(0)

    @pl.when(i == 0)
    def _():
        sums_ref[...] = jnp.zeros_like(sums_ref)
        cnts_ref[...] = jnp.zeros_like(cnts_ref)

    vp = -jnp.log(EPS + jax.nn.sigmoid(tp_ref[...]))
    vn = -jnp.log(EPS + 1.0 - jax.nn.sigmoid(tn_ref[...]))
    gp = (sp_ref[...] * G) // N
    gn = (sn_ref[...] * G) // N

    ps, ns, pc, nc = [], [], [], []
    for g in range(G):
        mp = (gp == g)
        mn = (gn == g)
        ps.append(jnp.sum(jnp.where(mp, vp, 0.0)))
        ns.append(jnp.sum(jnp.where(mn, vn, 0.0)))
        pc.append(jnp.sum(mp.astype(jnp.float32)))
        nc.append(jnp.sum(mn.astype(jnp.float32)))
    sums_ref[...] += jnp.stack([jnp.stack(ps), jnp.stack(ns)])
    cnts_ref[...] += jnp.stack([jnp.stack(pc), jnp.stack(nc)])

    @pl.when(i == GRID_E - 1)
    def _():
        s = sums_ref[...]
        c = jnp.maximum(cnts_ref[...], 1.0)
        lrc_ref[...] = (jnp.sum(s / c) / G).reshape(1, 1)


def _tc_loss(tpos, tneg, spos, sneg):
    return pl.pallas_call(
        _loss_body,
        grid=(GRID_E,),
        in_specs=[
            pl.BlockSpec((EBR, 128), lambda i: (i, 0)),
            pl.BlockSpec((EBR, 128), lambda i: (i, 0)),
            pl.BlockSpec((EBR, 128), lambda i: (i, 0)),
            pl.BlockSpec((EBR, 128), lambda i: (i, 0)),
        ],
        out_specs=[
            pl.BlockSpec((2, G), lambda i: (0, 0)),
            pl.BlockSpec((2, G), lambda i: (0, 0)),
            pl.BlockSpec((1, 1), lambda i: (0, 0)),
        ],
        out_shape=[
            jax.ShapeDtypeStruct((2, G), jnp.float32),
            jax.ShapeDtypeStruct((2, G), jnp.float32),
            jax.ShapeDtypeStruct((1, 1), jnp.float32),
        ],
    )(tpos.reshape(EB, 128), tneg.reshape(EB, 128),
      spos.reshape(EB, 128), sneg.reshape(EB, 128))


def _head_body(xs_ref, w1_ref, b1_ref, w2_ref, b2_ref,
               preds_ref, acc_ref, cnt_ref):
    i = pl.program_id(0)

    @pl.when(i == 0)
    def _():
        acc_ref[...] = jnp.zeros_like(acc_ref)
        cnt_ref[...] = jnp.zeros_like(cnt_ref)

    hc = jnp.dot(xs_ref[...], w1_ref[...], preferred_element_type=jnp.float32)
    hc = hc + b1_ref[...]
    hc = jnp.where(hc > 0, hc, 0.01 * hc)
    hc = jnp.dot(hc, w2_ref[...], preferred_element_type=jnp.float32) + b2_ref[...]
    tmp = jnp.exp(hc - jnp.max(hc, axis=-1, keepdims=True)) + EPS
    preds = tmp / jnp.sum(tmp, axis=-1, keepdims=True)
    preds_ref[...] = preds

    rows = i * BR + lax.broadcasted_iota(jnp.int32, (BR, 1), 0)
    gcol = (rows * G) // N
    onehot = (gcol == lax.broadcasted_iota(jnp.int32, (1, G), 1)).astype(jnp.float32)
    acc_ref[...] += lax.dot_general(onehot, preds, (((0,), (0,)), ((), ())),
                                    preferred_element_type=jnp.float32)
    cnt_ref[...] += lax.dot_general(onehot, jnp.ones((BR, 1), jnp.float32),
                                    (((0,), (0,)), ((), ())),
                                    preferred_element_type=jnp.float32)

    @pl.when(i == GRID_N - 1)
    def _():
        acc_ref[...] = jnp.log(acc_ref[...] / jnp.maximum(cnt_ref[...], 1.0))


def _tc_head(xs, W1, b1, W2, b2):
    HH = HEADS * HID
    logyp, _, preds = None, None, None
    outs = pl.pallas_call(
        _head_body,
        grid=(GRID_N,),
        in_specs=[
            pl.BlockSpec((BR, HH), lambda i: (i, 0)),
            pl.BlockSpec((HH, HID), lambda i: (0, 0)),
            pl.BlockSpec((1, HID), lambda i: (0, 0)),
            pl.BlockSpec((HID, NCLS), lambda i: (0, 0)),
            pl.BlockSpec((1, NCLS), lambda i: (0, 0)),
        ],
        out_specs=[
            pl.BlockSpec((BR, NCLS), lambda i: (i, 0)),
            pl.BlockSpec((G, NCLS), lambda i: (0, 0)),
            pl.BlockSpec((G, 1), lambda i: (0, 0)),
        ],
        out_shape=[
            jax.ShapeDtypeStruct((N, NCLS), jnp.float32),
            jax.ShapeDtypeStruct((G, NCLS), jnp.float32),
            jax.ShapeDtypeStruct((G, 1), jnp.float32),
        ],
    )(xs, W1, b1.reshape(1, HID), W2, b2.reshape(1, NCLS))
    preds, logyp, _ = outs
    return logyp, preds


_SC_MESH = plsc.VectorSubcoreMesh(core_axis_name="c", subcore_axis_name="s")
NSC = 2
NTILE = 16
NW = NSC * NTILE
ECH = 400
EPT = E // NTILE
HR = N // NSC
TRASH = HR
RA = 312
ZR = 104
TAILB = 15 * RA + 3 * ZR


ACCR = HR + 8


def _zero_acc(s, zb_v, acc_sh, width):
    def zstep(i, _):
        for k in range(width // 16):
            zb_v[i, pl.ds(k * 16, 16)] = jnp.zeros((16,), jnp.float32)
        return 0

    lax.fori_loop(0, ZR, zstep, 0)
    for j in range(3):
        pltpu.sync_copy(zb_v, acc_sh.at[pl.ds(s * RA + j * ZR, ZR)])

    @pl.when(s == NTILE - 1)
    def _():
        pltpu.sync_copy(zb_v.at[pl.ds(0, 8)], acc_sh.at[pl.ds(TAILB, 8)])


def _local_dst(c, dst_v, dstl_v):
    def tgrp(k, _):
        d16 = dst_v[pl.ds(k * 16, 16)]
        loc = d16 - c * HR
        ok = (loc >= 0) & (loc < HR)
        dstl_v[pl.ds(k * 16, 16)] = jnp.where(ok, loc, TRASH)
        return 0

    lax.fori_loop(0, ECH // 16, tgrp, 0)


def _drain_acc(c, s, acc_sh, out_hbm):
    pltpu.sync_copy(acc_sh.at[pl.ds(s * RA, RA)],
                    out_hbm.at[pl.ds(c * HR + s * RA, RA)])

    @pl.when(s == NTILE - 1)
    def _():
        pltpu.sync_copy(acc_sh.at[pl.ds(TAILB, 8)],
                        out_hbm.at[pl.ds(c * HR + TAILB, 8)])


@functools.partial(
    pl.kernel, mesh=_SC_MESH,
    out_type=jax.ShapeDtypeStruct((N, D), jnp.float32),
    scratch_types=[
        pltpu.VMEM((ECH,), jnp.int32),
        pltpu.VMEM((ECH,), jnp.int32),
        pltpu.VMEM((ECH, D), jnp.float32),
        pltpu.VMEM((ZR, D), jnp.float32),
        pltpu.VMEM_SHARED((ACCR, D), jnp.float32),
    ],
)
def _sc_deg(dst_hbm, out_hbm, dst_v, dstl_v, ones_v, zb_v, acc_sh):
    c = lax.axis_index("c")
    s = lax.axis_index("s")

    def fstep(i, _):
        for k in range(D // 16):
            ones_v[i, pl.ds(k * 16, 16)] = jnp.ones((16,), jnp.float32)
        return 0

    lax.fori_loop(0, ECH, fstep, 0)
    _zero_acc(s, zb_v, acc_sh, D)
    plsc.subcore_barrier()

    def chunk(ci, _):
        base = s * EPT + ci * ECH
        pltpu.sync_copy(dst_hbm.at[pl.ds(base, ECH)], dst_v)
        _local_dst(c, dst_v, dstl_v)
        pltpu.sync_copy(ones_v, acc_sh.at[dstl_v], add=True)
        return 0

    lax.fori_loop(0, EPT // ECH, chunk, 0)
    plsc.subcore_barrier()
    _drain_acc(c, s, acc_sh, out_hbm)


@functools.partial(
    pl.kernel, mesh=_SC_MESH,
    out_type=jax.ShapeDtypeStruct((N, D), jnp.float32),
    scratch_types=[
        pltpu.VMEM((ECH,), jnp.int32),
        pltpu.VMEM((ECH,), jnp.int32),
        pltpu.VMEM((ECH,), jnp.int32),
        pltpu.VMEM((ECH, D), jnp.float32),
        pltpu.VMEM((ZR, D), jnp.float32),
        pltpu.VMEM_SHARED((ACCR, D), jnp.float32),
        pltpu.SemaphoreType.DMA,
    ],
)
def _sc_gcn(u_hbm, src_hbm, dst_hbm, out_hbm,
            src_v, dst_v, dstl_v, rows_v, zb_v, acc_sh, sem):
    c = lax.axis_index("c")
    s = lax.axis_index("s")
    _zero_acc(s, zb_v, acc_sh, D)
    plsc.subcore_barrier()

    def chunk(ci, _):
        base = s * EPT + ci * ECH
        pltpu.sync_copy(src_hbm.at[pl.ds(base, ECH)], src_v)
        pltpu.sync_copy(dst_hbm.at[pl.ds(base, ECH)], dst_v)
        _local_dst(c, dst_v, dstl_v)
        pltpu.async_copy(u_hbm.at[src_v], rows_v, sem).wait()
        pltpu.sync_copy(rows_v, acc_sh.at[dstl_v], add=True)
        return 0

    lax.fori_loop(0, EPT // ECH, chunk, 0)
    plsc.subcore_barrier()
    _drain_acc(c, s, acc_sh, out_hbm)


def _seg_scatter(u, src, dst):
    p = jax.ops.segment_sum(u[src], dst, num_segments=N)
    return p, jnp.zeros_like(p)


def _deg_partials(dst):
    d = jax.ops.segment_sum(jnp.ones((E,), jnp.float32), dst, num_segments=N)
    p = jnp.zeros((32, N), jnp.float32).at[0].set(d)
    return p


def _dec_dots(hx, s, d):
    return jnp.sum(hx[s] * hx[d], axis=1)


def _att_xs(mask, xh, a, b):
    mi = jnp.nonzero(mask, size=M_ATT, fill_value=0)
    midx0, midx1 = mi[0], mi[1]
    mvalid = (jnp.arange(M_ATT) < jnp.sum(mask)).astype(jnp.float32)
    s = a[midx0] + b[midx1]
    w = jax.nn.sigmoid(jnp.where(s > 0, s, 0.01 * s))
    xm = (xh[midx1].reshape(-1, HEADS, HID) * w[:, :, None]).reshape(-1, HEADS * HID)
    xm = xm * mvalid[:, None]
    return jax.ops.segment_sum(xm, midx0, num_segments=N)


def kernel(x, edge_index, neg_edge_index, batch, mask,
           W_fc, b_fc, W_conv0, b_conv0, gamma0, beta0,
           W_conv1, b_conv1, gamma1, beta1, W_conv2, b_conv2, gamma2, beta2,
           W_feat, W_fc1, b_fc1, W_fc2, b_fc2, phi):
    src, dst = edge_index[0], edge_index[1]
    nsrc, ndst = neg_edge_index[0], neg_edge_index[1]

    dis = _tc_deg(_sc_deg(dst)[:, 0].reshape(1, N)).reshape(N, 1)

    h, stats = _tc_fc(x, W_fc, b_fc)
    hs = []
    for (Wc, bc, gm, bt) in ((W_conv0, b_conv0, gamma0, beta0),
                             (W_conv1, b_conv1, gamma1, beta1),
                             (W_conv2, b_conv2, gamma2, beta2)):
        u = _tc_bnu(h, stats, gm, bt, dis)
        p = _sc_gcn(u, src, dst)
        h, stats = _tc_mm(p, u, dis, Wc, bc)
        hs.append(h)
    hx = jnp.concatenate(hs, axis=-1)

    pa = phi[:, :HID, 0]
    pb = phi[:, HID:, 0]
    eye = jnp.eye(HEADS, dtype=jnp.float32)
    PhiA = (eye[:, None, :] * pa[:, :, None]).reshape(HEADS * HID, HEADS)
    PhiB = (eye[:, None, :] * pb[:, :, None]).reshape(HEADS * HID, HEADS)
    xh, av, bv = _tc_feat(hx, W_feat, PhiA, PhiB)

    tpos = _dec_dots(hx, src, dst)
    tneg = _dec_dots(hx, nsrc, ndst)
    _, _, lrc = _tc_loss(tpos, tneg, src, nsrc)
    Lrc = lrc.reshape(())

    xs = _att_xs(mask, xh, av, bv)
    logyp, preds = _tc_head(xs, W_fc1, b_fc1, W_fc2, b_fc2)
    return (logyp, Lrc, preds)
